# baseline (speedup 1.0000x reference)
import functools

import jax
import jax.numpy as jnp
from jax.experimental import pallas as pl

N, D, K = 16384, 128, 1000
K_PAD = 1024
TN = 1024


def _kmeans_body(x_ref, c_ref, dist_ref, assign_ref):
    x = x_ref[...]
    c = c_ref[...]
    x2 = jnp.sum(x * x, axis=1, keepdims=True)
    c2 = jnp.sum(c * c, axis=1)[None, :]
    xc = jax.lax.dot_general(
        x, c, (((1,), (1,)), ((), ())),
        preferred_element_type=jnp.float32)
    sq = x2 + c2 - 2.0 * xc
    dist = jnp.sqrt(jnp.clip(sq, 1e-12))
    dist_ref[...] = dist[:, :K]
    col = jax.lax.broadcasted_iota(jnp.int32, (TN, K_PAD), 1)
    masked = jnp.where(col < K, dist, jnp.inf)
    assign_ref[...] = jnp.argmin(masked, axis=1).astype(jnp.int32)


@jax.jit
def kernel(data, centroids):
    c_pad = jnp.zeros((K_PAD, D), jnp.float32).at[:K].set(centroids)
    grid = (N // TN,)
    dist, assign = pl.pallas_call(
        _kmeans_body,
        grid=grid,
        in_specs=[
            pl.BlockSpec((TN, D), lambda i: (i, 0)),
            pl.BlockSpec((K_PAD, D), lambda i: (0, 0)),
        ],
        out_specs=[
            pl.BlockSpec((TN, K), lambda i: (i, 0)),
            pl.BlockSpec((TN,), lambda i: (i,)),
        ],
        out_shape=[
            jax.ShapeDtypeStruct((N, K), jnp.float32),
            jax.ShapeDtypeStruct((N,), jnp.int32),
        ],
    )(data, c_pad)
    return dist, assign
